# Initial kernel scaffold; baseline (speedup 1.0000x reference)
import jax
import jax.numpy as jnp
from jax.experimental import pallas as pl


def kernel(x, edge_index, edge_type, W_msg, b_msg, W_ih, W_hh, b_ih, b_hh, W1, b1, W2, b2, W3, b3):
    raise NotImplementedError("write your pallas kernel here")


# speedup vs baseline: 8.8914x; 8.8914x over previous
import functools

import jax
import jax.numpy as jnp
from jax import lax
from jax.experimental import pallas as pl
from jax.experimental.pallas import tpu as pltpu
from jax.experimental.pallas import tpu_sc as plsc

N = 10000
D = 20
DP = 32
NP = 10240
E = 160000
NC, NS = 2, 16
NW = NC * NS
EPAD = 163840
EW = EPAD // NW
CH = 40
CSZ = 128
RPS = NP // NS

_HIGH = lax.Precision.HIGHEST


def _sc_aggregate_body(h_hbm, src_hbm, dst_hbm, zeros_hbm, out_hbm,
                       src_v, dst_v, rows_v, s_sh, sem):
    c = lax.axis_index("c")
    s = lax.axis_index("s")
    wid = s * NC + c
    pltpu.sync_copy(zeros_hbm.at[pl.ds(s * RPS, RPS)],
                    s_sh.at[pl.ds(s * RPS, RPS)])
    pltpu.sync_copy(src_hbm.at[wid], src_v)
    pltpu.sync_copy(dst_hbm.at[wid], dst_v)
    plsc.subcore_barrier()

    def chunk(j, carry):
        pltpu.async_copy(h_hbm.at[src_v.at[j]], rows_v, sem).wait()
        pltpu.sync_copy(rows_v, s_sh.at[dst_v.at[j]], add=True)
        return carry

    lax.fori_loop(0, CH, chunk, 0)
    plsc.subcore_barrier()
    pltpu.sync_copy(s_sh.at[pl.ds(s * RPS, RPS)],
                    out_hbm.at[c, pl.ds(s * RPS, RPS)])


@functools.lru_cache(maxsize=None)
def _sc_aggregate():
    return pl.kernel(
        _sc_aggregate_body,
        out_type=jax.ShapeDtypeStruct((NC, NP, DP), jnp.float32),
        mesh=plsc.VectorSubcoreMesh(core_axis_name="c", subcore_axis_name="s",
                                    num_cores=NC, num_subcores=NS),
        scratch_types=[
            pltpu.VMEM((CH, CSZ), jnp.int32),
            pltpu.VMEM((CH, CSZ), jnp.int32),
            pltpu.VMEM((CSZ, DP), jnp.float32),
            pltpu.VMEM_SHARED((NP, DP), jnp.float32),
            pltpu.SemaphoreType.DMA,
        ],
        compiler_params=pltpu.CompilerParams(use_tc_tiling_on_sc=False),
    )


def _rq(v):
    return v.astype(jnp.bfloat16).astype(jnp.float32)


def _gru_body(s2_ref, h_ref, w_ref, b_ref, out_ref, outq_ref):
    h = h_ref[...]
    S = s2_ref[0] + s2_ref[1] + _rq(h)
    deg = S[:, 20:21]
    dotT = lambda x, w: lax.dot_general(x, w, (((1,), (1,)), ((), ())))
    a = lax.dot_general(S, w_ref[0], (((1,), (1,)), ((), ())),
                        precision=_HIGH) + deg * b_ref[0:1]
    gi_r = dotT(a, w_ref[1]) + b_ref[1:2]
    gi_z = dotT(a, w_ref[2]) + b_ref[2:3]
    gi_n = dotT(a, w_ref[3]) + b_ref[3:4]
    gh_r = dotT(h, w_ref[4]) + b_ref[4:5]
    gh_z = dotT(h, w_ref[5]) + b_ref[5:6]
    gh_n = dotT(h, w_ref[6]) + b_ref[6:7]
    r = jax.nn.sigmoid(gi_r + gh_r)
    z = jax.nn.sigmoid(gi_z + gh_z)
    n = jnp.tanh(gi_n + r * gh_n)
    hn = jax.nn.relu((1.0 - z) * n + z * h)
    lane = lax.broadcasted_iota(jnp.int32, hn.shape, 1)
    hout = jnp.where(lane < D, hn, jnp.where(lane == D, 1.0, 0.0))
    out_ref[...] = hout
    outq_ref[...] = _rq(hout)


_BR = 2048

_gru_call = pl.pallas_call(
    _gru_body,
    grid=(NP // _BR,),
    in_specs=[
        pl.BlockSpec((NC, _BR, DP), lambda i: (0, i, 0)),
        pl.BlockSpec((_BR, DP), lambda i: (i, 0)),
        pl.BlockSpec((7, DP, DP), lambda i: (0, 0, 0)),
        pl.BlockSpec((7, DP), lambda i: (0, 0)),
    ],
    out_specs=[pl.BlockSpec((_BR, DP), lambda i: (i, 0)),
               pl.BlockSpec((_BR, DP), lambda i: (i, 0))],
    out_shape=[jax.ShapeDtypeStruct((NP, DP), jnp.float32),
               jax.ShapeDtypeStruct((NP, DP), jnp.float32)],
)


def _mlp_body(h_ref, w_ref, b_ref, out_ref):
    dotT = lambda x, w: lax.dot_general(x, w, (((1,), (1,)), ((), ())))
    u = h_ref[...]
    lane = lax.broadcasted_iota(jnp.int32, u.shape, 1)
    u = jnp.where(lane < D, u, 0.0)
    u = jax.nn.relu(dotT(u, w_ref[0]) + b_ref[0:1])
    u = jax.nn.relu(dotT(u, w_ref[1]) + b_ref[1:2])
    u = jax.nn.relu(dotT(u, w_ref[2]) + b_ref[2:3])
    out_ref[...] = u


_mlp_call = pl.pallas_call(
    _mlp_body,
    grid=(NP // _BR,),
    in_specs=[
        pl.BlockSpec((_BR, DP), lambda i: (i, 0)),
        pl.BlockSpec((3, DP, DP), lambda i: (0, 0, 0)),
        pl.BlockSpec((3, DP), lambda i: (0, 0)),
    ],
    out_specs=pl.BlockSpec((_BR, DP), lambda i: (i, 0)),
    out_shape=jax.ShapeDtypeStruct((NP, DP), jnp.float32),
)

_BM = 400


def _gram_body(ui_ref, uall_ref, out_ref):
    out_ref[...] = lax.dot_general(ui_ref[...], uall_ref[:N, :],
                                   (((1,), (1,)), ((), ())))


_gram_call = pl.pallas_call(
    _gram_body,
    grid=(N // _BM,),
    in_specs=[
        pl.BlockSpec((_BM, DP), lambda i: (i, 0)),
        pl.BlockSpec((NP, DP), lambda i: (0, 0)),
    ],
    out_specs=pl.BlockSpec((_BM, N), lambda i: (i, 0)),
    out_shape=jax.ShapeDtypeStruct((N, N), jnp.float32),
)


def kernel(x, edge_index, edge_type, W_msg, b_msg, W_ih, W_hh, b_ih, b_hh,
           W1, b1, W2, b2, W3, b3):
    f32 = jnp.float32
    n_conv = W_msg.shape[0]

    src = jnp.concatenate([edge_index[0],
                           jnp.full((EPAD - E,), N, jnp.int32)]).reshape(NW, CH, CSZ)
    dst = jnp.concatenate([edge_index[1],
                           jnp.full((EPAD - E,), N, jnp.int32)]).reshape(NW, CH, CSZ)

    h = jnp.zeros((NP, DP), f32)
    h = h.at[:N, :x.shape[1]].set(x.astype(f32))
    h = h.at[:, D].set(1.0)

    def padw(w):
        return jnp.zeros((DP, DP), f32).at[:w.shape[0], :w.shape[1]].set(w)

    def padb(b):
        return jnp.zeros((DP,), f32).at[:b.shape[0]].set(b)

    wl, bl = [], []
    for i in range(n_conv):
        ihr, ihz, ihn = W_ih[i, :D], W_ih[i, D:2 * D], W_ih[i, 2 * D:]
        hhr, hhz, hhn = W_hh[i, :D], W_hh[i, D:2 * D], W_hh[i, 2 * D:]
        wmq = W_msg[i].astype(jnp.bfloat16).astype(f32)
        wl.append(jnp.stack([padw(wmq), padw(ihr), padw(ihz), padw(ihn),
                             padw(hhr), padw(hhz), padw(hhn)]))
        bl.append(jnp.stack([padb(b_msg[i]),
                             padb(b_ih[i, :D]), padb(b_ih[i, D:2 * D]),
                             padb(b_ih[i, 2 * D:]),
                             padb(b_hh[i, :D]), padb(b_hh[i, D:2 * D]),
                             padb(b_hh[i, 2 * D:])]))
    wmlp = jnp.stack([padw(W1), padw(W2), padw(W3)])
    bmlp = jnp.stack([padb(b1), padb(b2), padb(b3)])
    zeros = jnp.zeros((NP, DP), f32)

    hq = h.astype(jnp.bfloat16).astype(f32)
    for i in range(n_conv):
        s2 = _sc_aggregate()(hq, src, dst, zeros)
        h, hq = _gru_call(s2, h, wl[i], bl[i])

    u = _mlp_call(h, wmlp, bmlp)
    g = _gram_call(u, u)
    return g[None]
